# baseline (speedup 1.0000x reference)
import jax
import jax.numpy as jnp
from jax.experimental import pallas as pl

B = 16384
UNI = 128
HID = 64
HEADS = 4
C1 = HID // HEADS
BT = 2048


def _leaky(x):
    return jnp.where(x >= 0, x, 0.2 * x)


def _elu(x):
    return jnp.where(x > 0, x, jnp.exp(x) - 1.0)


def _gnn_kernel(t_ref, a_ref, v_ref,
                gl0_ref, gl1_ref, gl2_ref, gr0_ref, gr1_ref, gr2_ref,
                cl_ref, cr_ref, ma1_ref, bias1_ref,
                wl2_ref, bl2_ref, wr2_ref, br2_ref, ma2_ref, bias2_ref,
                out_ref):
    f32 = jnp.float32
    feats = (t_ref[...], a_ref[...], v_ref[...])
    gl = (gl0_ref[...], gl1_ref[...], gl2_ref[...])
    gr = (gr0_ref[...], gr1_ref[...], gr2_ref[...])
    cl = cl_ref[...]
    cr = cr_ref[...]

    xl = [jnp.dot(feats[i], gl[i], preferred_element_type=f32) + cl[i:i + 1]
          for i in range(3)]
    xr = [jnp.dot(feats[i], gr[i], preferred_element_type=f32) + cr[i:i + 1]
          for i in range(3)]

    ma1 = ma1_ref[...]
    bias1 = bias1_ref[...]

    def gat(xli, xri, ma, bias):
        outs = []
        for d in range(3):
            a, b = [s for s in range(3) if s != d]
            ea = _leaky(xli[a] + xri[d])
            eb = _leaky(xli[b] + xri[d])
            dlog = jnp.dot(eb - ea, ma, preferred_element_type=f32)
            sa = 1.0 / (1.0 + jnp.exp(dlog))
            agg = xli[b] + sa * (xli[a] - xli[b])
            outs.append(agg + bias)
        return outs

    h = [_elu(o) for o in gat(xl, xr, ma1, bias1)]

    wl2 = wl2_ref[...]
    wr2 = wr2_ref[...]
    bl2 = bl2_ref[...]
    br2 = br2_ref[...]
    yl = [jnp.dot(h[i], wl2, preferred_element_type=f32) + bl2 for i in range(3)]
    yr = [jnp.dot(h[i], wr2, preferred_element_type=f32) + br2 for i in range(3)]
    o2 = gat(yl, yr, ma2_ref[...], bias2_ref[...])
    out_ref[...] = (o2[0] + o2[1] + o2[2]) * (1.0 / 3.0)


@jax.jit
def kernel(text_features, audio_features, video_features, W_text, b_text,
           W_audio, b_audio, W_video, b_video, Wl1, bl1, Wr1, br1, att1,
           bias1, Wl2, bl2, Wr2, br2, att2, bias2):
    f32 = jnp.float32
    row = lambda v: v.reshape(1, -1).astype(f32)

    Ws = (W_text, W_audio, W_video)
    bs = (b_text, b_audio, b_video)
    gls = [(W.T @ Wl1.T).astype(f32) for W in Ws]
    grs = [(W.T @ Wr1.T).astype(f32) for W in Ws]
    cl = jnp.stack([b @ Wl1.T + bl1 for b in bs]).astype(f32)
    cr = jnp.stack([b @ Wr1.T + br1 for b in bs]).astype(f32)

    att1_flat = att1.reshape(HEADS * C1)
    head = jnp.arange(HID, dtype=jnp.int32) // C1
    same = (head[:, None] == head[None, :]).astype(f32)
    ma1 = (att1_flat[:, None] * same).astype(f32)
    ma2 = jnp.broadcast_to(att2.reshape(HID, 1), (HID, HID)).astype(f32)

    grid = (B // BT,)
    data_spec = pl.BlockSpec((BT, UNI), lambda i: (i, 0))
    w_uni = pl.BlockSpec((UNI, HID), lambda i: (0, 0))
    w_hid = pl.BlockSpec((HID, HID), lambda i: (0, 0))
    c3 = pl.BlockSpec((3, HID), lambda i: (0, 0))
    vec = pl.BlockSpec((1, HID), lambda i: (0, 0))

    out = pl.pallas_call(
        _gnn_kernel,
        grid=grid,
        in_specs=[
            data_spec, data_spec, data_spec,
            w_uni, w_uni, w_uni, w_uni, w_uni, w_uni,
            c3, c3, w_hid, vec,
            w_hid, vec, w_hid, vec, w_hid, vec,
        ],
        out_specs=pl.BlockSpec((BT, HID), lambda i: (i, 0)),
        out_shape=jax.ShapeDtypeStruct((B, HID), f32),
    )(
        text_features, audio_features, video_features,
        gls[0], gls[1], gls[2], grs[0], grs[1], grs[2],
        cl, cr, ma1, row(bias1),
        Wl2.T.astype(f32), row(bl2), Wr2.T.astype(f32), row(br2),
        ma2, row(bias2),
    )
    return out


# speedup vs baseline: 1.2012x; 1.2012x over previous
import jax
import jax.numpy as jnp
from jax import lax
from jax.experimental import pallas as pl

B = 16384
UNI = 128
HID = 64
HEADS = 4
C1 = HID // HEADS
BT = 1024

_TRANS_RHS = (((1,), (1,)), ((), ()))
_TRANS_LHS = (((0,), (1,)), ((), ()))


def _leaky(x):
    return jnp.where(x >= 0, x, 0.2 * x)


def _elu(x):
    return jnp.where(x > 0, x, jnp.exp(x) - 1.0)


def _gnn_kernel(t_ref, a_ref, v_ref,
                wt_ref, wa_ref, wv_ref,
                wl1_ref, wr1_ref, att1_ref,
                wl2_ref, wr2_ref, att2_ref,
                bvec_ref, out_ref):
    f32 = jnp.float32

    def dg(x, y, dims):
        return lax.dot_general(x, y, dims, preferred_element_type=f32)

    feats = (t_ref[...], a_ref[...], v_ref[...])
    wn = (wt_ref[...], wa_ref[...], wv_ref[...])
    wl1 = wl1_ref[...]
    wr1 = wr1_ref[...]
    wl2 = wl2_ref[...]
    wr2 = wr2_ref[...]
    bv = bvec_ref[...]
    bn = [bv[i:i + 1] for i in range(3)]
    bl1, br1, bias1 = bv[3:4], bv[4:5], bv[5:6]
    bl2, br2, bias2 = bv[6:7], bv[7:8], bv[8:9]

    rh = lax.broadcasted_iota(jnp.int32, (HID, HID), 0)
    ch = lax.broadcasted_iota(jnp.int32, (HID, HID), 1)
    ident = (rh == ch).astype(f32)
    att1_col = dg(ident, att1_ref[...], _TRANS_RHS)
    ma1 = att1_col * (rh // C1 == ch // C1).astype(f32)

    xl, xr = [], []
    for i in range(3):
        ql = dg(wn[i], wl1, _TRANS_LHS)
        qr = dg(wn[i], wr1, _TRANS_LHS)
        cl = dg(bn[i], wl1, _TRANS_RHS) + bl1
        cr = dg(bn[i], wr1, _TRANS_RHS) + br1
        xl.append(jnp.dot(feats[i], ql, preferred_element_type=f32) + cl)
        xr.append(jnp.dot(feats[i], qr, preferred_element_type=f32) + cr)

    def gat(xli, xri, logit_mat, dims):
        outs = []
        for d in range(3):
            a, b = [s for s in range(3) if s != d]
            ea = _leaky(xli[a] + xri[d])
            eb = _leaky(xli[b] + xri[d])
            dlog = dg(eb - ea, logit_mat, dims)
            sa = 1.0 / (1.0 + jnp.exp(dlog))
            outs.append(xli[b] + sa * (xli[a] - xli[b]))
        return outs

    plain = (((1,), (0,)), ((), ()))
    h = [_elu(o + bias1) for o in gat(xl, xr, ma1, plain)]

    yl = [dg(h[i], wl2, _TRANS_RHS) + bl2 for i in range(3)]
    yr = [dg(h[i], wr2, _TRANS_RHS) + br2 for i in range(3)]
    o2 = gat(yl, yr, att2_ref[...], _TRANS_RHS)
    out_ref[...] = (o2[0] + o2[1] + o2[2]) * (1.0 / 3.0) + bias2


@jax.jit
def kernel(text_features, audio_features, video_features, W_text, b_text,
           W_audio, b_audio, W_video, b_video, Wl1, bl1, Wr1, br1, att1,
           bias1, Wl2, bl2, Wr2, br2, att2, bias2):
    f32 = jnp.float32
    bvec = jnp.stack([b_text, b_audio, b_video, bl1, br1, bias1,
                      bl2, br2, bias2]).astype(f32)
    att1_row = att1.reshape(1, HEADS * C1).astype(f32)

    grid = (B // BT,)
    data_spec = pl.BlockSpec((BT, UNI), lambda i: (i, 0))
    w_proj = pl.BlockSpec((HID, UNI), lambda i: (0, 0))
    w_hid = pl.BlockSpec((HID, HID), lambda i: (0, 0))
    vec = pl.BlockSpec((1, HID), lambda i: (0, 0))

    out = pl.pallas_call(
        _gnn_kernel,
        grid=grid,
        in_specs=[
            data_spec, data_spec, data_spec,
            w_proj, w_proj, w_proj,
            w_hid, w_hid, vec,
            w_hid, w_hid, vec,
            pl.BlockSpec((9, HID), lambda i: (0, 0)),
        ],
        out_specs=pl.BlockSpec((BT, HID), lambda i: (i, 0)),
        out_shape=jax.ShapeDtypeStruct((B, HID), f32),
    )(
        text_features, audio_features, video_features,
        W_text, W_audio, W_video,
        Wl1, Wr1, att1_row,
        Wl2, Wr2, att2,
        bvec,
    )
    return out


# speedup vs baseline: 1.3186x; 1.0977x over previous
import jax
import jax.numpy as jnp
from jax import lax
from jax.experimental import pallas as pl
from jax.experimental.pallas import tpu as pltpu

B = 16384
UNI = 128
HID = 64
HEADS = 4
C1 = HID // HEADS
BT = 1024

_TRANS_RHS = (((1,), (1,)), ((), ()))
_TRANS_LHS = (((0,), (1,)), ((), ()))
_PLAIN = (((1,), (0,)), ((), ()))


def _leaky(x):
    return jnp.maximum(x, 0.2 * x)


def _elu(x):
    return jnp.where(x > 0, x, jnp.exp(x) - 1.0)


def _gnn_kernel(t_ref, a_ref, v_ref,
                wt_ref, wa_ref, wv_ref,
                wl1_ref, wr1_ref, att1_ref,
                wl2_ref, wr2_ref, att2_ref,
                bvec_ref, out_ref,
                q_ref, c_ref, w2_ref, b2_ref, ma1_ref):
    f32 = jnp.float32

    def dg(x, y, dims):
        return lax.dot_general(x, y, dims, preferred_element_type=f32)

    @pl.when(pl.program_id(0) == 0)
    def _prep():
        bv = bvec_ref[...]
        bn = [bv[i:i + 1] for i in range(3)]
        bl1, br1 = bv[3:4], bv[4:5]
        bl2, br2 = bv[6:7], bv[7:8]
        wl1 = wl1_ref[...]
        wr1 = wr1_ref[...]
        wn = (wt_ref[...], wa_ref[...], wv_ref[...])
        for i in range(3):
            q_ref[i, :, :HID] = dg(wn[i], wl1, _TRANS_LHS)
            q_ref[i, :, HID:] = dg(wn[i], wr1, _TRANS_LHS)
            c_ref[i:i + 1, :HID] = dg(bn[i], wl1, _TRANS_RHS) + bl1
            c_ref[i:i + 1, HID:] = dg(bn[i], wr1, _TRANS_RHS) + br1
        rh = lax.broadcasted_iota(jnp.int32, (HID, HID), 0)
        ch = lax.broadcasted_iota(jnp.int32, (HID, HID), 1)
        ident = (rh == ch).astype(f32)
        w2_ref[:, :HID] = dg(ident, wl2_ref[...], _TRANS_RHS)
        w2_ref[:, HID:] = dg(ident, wr2_ref[...], _TRANS_RHS)
        b2_ref[0:1, :HID] = bl2
        b2_ref[0:1, HID:] = br2
        att1_col = dg(ident, att1_ref[...], _TRANS_RHS)
        ma1_ref[...] = att1_col * (rh // C1 == ch // C1).astype(f32)

    feats = (t_ref[...], a_ref[...], v_ref[...])
    bv = bvec_ref[...]
    bias1, bias2 = bv[5:6], bv[8:9]
    ma1 = ma1_ref[...]
    c = c_ref[...]

    xlr = [jnp.dot(feats[i], q_ref[i], preferred_element_type=f32) + c[i:i + 1]
           for i in range(3)]

    def gat(packs, logit_mat, dims):
        xls = [p[:, :HID] for p in packs]
        xrs = [p[:, HID:] for p in packs]
        outs = []
        for d in range(3):
            a, b = [s for s in range(3) if s != d]
            ea = _leaky(xls[a] + xrs[d])
            eb = _leaky(xls[b] + xrs[d])
            dlog = dg(eb - ea, logit_mat, dims)
            sa = 1.0 / (1.0 + jnp.exp(dlog))
            outs.append(xls[b] + sa * (xls[a] - xls[b]))
        return outs

    h = [_elu(o + bias1) for o in gat(xlr, ma1, _PLAIN)]

    b2 = b2_ref[...]
    ylr = [jnp.dot(h[i], w2_ref[...], preferred_element_type=f32) + b2
           for i in range(3)]
    o2 = gat(ylr, att2_ref[...], _TRANS_RHS)
    out_ref[...] = (o2[0] + o2[1] + o2[2]) * (1.0 / 3.0) + bias2


@jax.jit
def kernel(text_features, audio_features, video_features, W_text, b_text,
           W_audio, b_audio, W_video, b_video, Wl1, bl1, Wr1, br1, att1,
           bias1, Wl2, bl2, Wr2, br2, att2, bias2):
    f32 = jnp.float32
    bvec = jnp.stack([b_text, b_audio, b_video, bl1, br1, bias1,
                      bl2, br2, bias2]).astype(f32)
    att1_row = att1.reshape(1, HEADS * C1).astype(f32)

    grid = (B // BT,)
    data_spec = pl.BlockSpec((BT, UNI), lambda i: (i, 0))
    w_proj = pl.BlockSpec((HID, UNI), lambda i: (0, 0))
    w_hid = pl.BlockSpec((HID, HID), lambda i: (0, 0))
    vec = pl.BlockSpec((1, HID), lambda i: (0, 0))

    out = pl.pallas_call(
        _gnn_kernel,
        grid=grid,
        in_specs=[
            data_spec, data_spec, data_spec,
            w_proj, w_proj, w_proj,
            w_hid, w_hid, vec,
            w_hid, w_hid, vec,
            pl.BlockSpec((9, HID), lambda i: (0, 0)),
        ],
        out_specs=pl.BlockSpec((BT, HID), lambda i: (i, 0)),
        out_shape=jax.ShapeDtypeStruct((B, HID), f32),
        scratch_shapes=[
            pltpu.VMEM((3, UNI, 2 * HID), f32),
            pltpu.VMEM((3, 2 * HID), f32),
            pltpu.VMEM((HID, 2 * HID), f32),
            pltpu.VMEM((1, 2 * HID), f32),
            pltpu.VMEM((HID, HID), f32),
        ],
    )(
        text_features, audio_features, video_features,
        W_text, W_audio, W_video,
        Wl1, Wr1, att1_row,
        Wl2, Wr2, att2,
        bvec,
    )
    return out
